# Initial kernel scaffold; baseline (speedup 1.0000x reference)
import jax
import jax.numpy as jnp
from jax.experimental import pallas as pl


def kernel(A_edge_indices, A_edge_values, X, target_x, target, conv_weights, gcn_weight, gcn_bias, lin_weight, lin_bias):
    raise NotImplementedError("write your pallas kernel here")


# speedup vs baseline: 2.8692x; 2.8692x over previous
import functools

import jax
import jax.numpy as jnp
from jax import lax
from jax.experimental import pallas as pl
from jax.experimental.pallas import tpu as pltpu
from jax.experimental.pallas import tpu_sc as plsc

N = 2048
NT = 4
E = 32768
QR = 256
NQ = N // QR
NSC = 2
NTILE = 16
EPT = E // NTILE
PANEL = QR * N
SLICE = PANEL // NTILE

W_IN = 256
W_OUT = 128
NCH = 2
NCLS = 4
NTGT = 1024
KB = 256
NKB = N // KB


def _adj_body(r_hbm, c_hbm, v_hbm, out_hbm,
              re, ce, ve, idxb, valb, zbuf, acc):
    cid = lax.axis_index("c")
    sid = lax.axis_index("s")

    def zb(i, carry):
        zbuf[pl.ds(i * 16, 16)] = jnp.zeros((16,), jnp.float32)
        return carry

    lax.fori_loop(0, SLICE // 16, zb, 0)

    rounds_per_type = NQ // NSC
    for rd in range(NT * rounds_per_type):
        t = rd // rounds_per_type
        q = NSC * (rd % rounds_per_type) + cid
        lo = q * QR

        pltpu.sync_copy(zbuf, acc.at[pl.ds(sid * SLICE, SLICE)])

        if rd % rounds_per_type == 0:
            ebase = t * E + sid * EPT
            pltpu.sync_copy(r_hbm.at[pl.ds(ebase, EPT)], re)
            pltpu.sync_copy(c_hbm.at[pl.ds(ebase, EPT)], ce)
            pltpu.sync_copy(v_hbm.at[pl.ds(ebase, EPT)], ve)

        def body(i, carry):
            sl = pl.ds(i * 16, 16)
            r = re[sl]
            c = ce[sl]
            v = ve[sl]
            inq = (r >= lo) & (r < lo + QR)
            idxb[sl] = ((r & (QR - 1)) << 11) + c
            valb[sl] = jnp.where(inq, v, jnp.zeros_like(v))
            return carry

        lax.fori_loop(0, EPT // 16, body, 0)

        plsc.subcore_barrier()
        pltpu.sync_copy(valb, acc.at[idxb], add=True)
        plsc.subcore_barrier()

        base = (t * NQ + q) * PANEL + sid * SLICE
        pltpu.sync_copy(acc.at[pl.ds(sid * SLICE, SLICE)],
                        out_hbm.at[pl.ds(base, SLICE)])


@functools.cache
def _build_adjacency_call():
    mesh = plsc.VectorSubcoreMesh(core_axis_name="c", subcore_axis_name="s",
                                  num_cores=NSC, num_subcores=NTILE)
    return pl.kernel(
        _adj_body,
        out_type=jax.ShapeDtypeStruct((NT * N * N,), jnp.float32),
        mesh=mesh,
        scratch_types=[
            pltpu.VMEM((EPT,), jnp.int32),
            pltpu.VMEM((EPT,), jnp.int32),
            pltpu.VMEM((EPT,), jnp.float32),
            pltpu.VMEM((EPT,), jnp.int32),
            pltpu.VMEM((EPT,), jnp.float32),
            pltpu.VMEM((SLICE,), jnp.float32),
            pltpu.VMEM_SHARED((PANEL,), jnp.float32),
        ],
    )


def _xw_body(x_ref, w_ref, out_ref):
    xw = jnp.dot(x_ref[...], w_ref[...], preferred_element_type=jnp.float32)
    out_ref[:, 0:W_OUT] = xw
    i = lax.broadcasted_iota(jnp.int32, (N, 128), 1)
    out_ref[:, W_OUT:W_OUT + 128] = jnp.where(i == 0, 1.0, 0.0)


_xw_call = pl.pallas_call(
    _xw_body,
    out_shape=jax.ShapeDtypeStruct((N, 256), jnp.float32),
)


def _chain_body(filt_ref, a_ref, xw_ref, out_ref, buf0, buf1):
    s = pl.program_id(0)
    k = pl.program_id(1)
    ksl = pl.ds(k * KB, KB)

    def stage(si, in_slice_fn, store):
        for c in range(NCH):
            m = None
            for e in range(NT):
                term = a_ref[e] * filt_ref[si, c, e]
                m = term if m is None else m + term
            p = jnp.dot(m, in_slice_fn(c), preferred_element_type=jnp.float32)
            store(c, p)

    @pl.when(s == 0)
    def _():
        def store(c, p):
            @pl.when(k == 0)
            def _():
                buf0[c] = p

            @pl.when(k != 0)
            def _():
                buf0[c] = buf0[c] + p

        stage(0, lambda c: xw_ref[ksl, :], store)

    @pl.when(s == 1)
    def _():
        def store(c, p):
            @pl.when(k == 0)
            def _():
                buf1[c] = p

            @pl.when(k != 0)
            def _():
                buf1[c] = buf1[c] + p

        stage(1, lambda c: buf0[c, ksl, :], store)

    @pl.when(s == 2)
    def _():
        def store(c, p):
            @pl.when(k == 0)
            def _():
                out_ref[c] = p

            @pl.when(k != 0)
            def _():
                out_ref[c] = out_ref[c] + p

        stage(2, lambda c: buf1[c, ksl, :], store)


_chain_call = pl.pallas_call(
    _chain_body,
    grid=(3, NKB),
    in_specs=[
        pl.BlockSpec(memory_space=pltpu.SMEM),
        pl.BlockSpec((NT, N, KB), lambda s, k: (0, 0, k)),
        pl.BlockSpec((N, 256), lambda s, k: (0, 0)),
    ],
    out_specs=pl.BlockSpec((NCH, N, 256), lambda s, k: (0, 0, 0)),
    out_shape=jax.ShapeDtypeStruct((NCH, N, 256), jnp.float32),
    scratch_shapes=[
        pltpu.VMEM((NCH, N, 256), jnp.float32),
        pltpu.VMEM((NCH, N, 256), jnp.float32),
    ],
    compiler_params=pltpu.CompilerParams(
        dimension_semantics=("arbitrary", "arbitrary")),
)


def _fin_body(v_ref, bias_ref, lw_ref, lb_ref, tgt_ref, out_ref):
    logits = None
    for c in range(NCH):
        vc = v_ref[c]
        deg = vc[:, W_OUT:W_OUT + 1]
        winv = jnp.where(deg == 0.0, 0.0, 1.0 / deg)
        xc = jnp.maximum(vc[:, 0:W_OUT] * winv + bias_ref[...], 0.0)
        lc = jnp.dot(xc, lw_ref[c * W_OUT:(c + 1) * W_OUT, :],
                     preferred_element_type=jnp.float32)
        logits = lc if logits is None else logits + lc
    tgt = tgt_ref[...]
    oh = (tgt == lax.broadcasted_iota(jnp.int32, (NTGT, N), 1))
    y = jnp.dot(oh.astype(jnp.float32), logits,
                preferred_element_type=jnp.float32)
    out_ref[...] = y + lb_ref[...]


_fin_call = pl.pallas_call(
    _fin_body,
    out_shape=jax.ShapeDtypeStruct((NTGT, NCLS), jnp.float32),
)


def kernel(A_edge_indices, A_edge_values, X, target_x, target, conv_weights,
           gcn_weight, gcn_bias, lin_weight, lin_bias):
    r_flat = A_edge_indices[:, 0, :].astype(jnp.int32).reshape(-1)
    c_flat = A_edge_indices[:, 1, :].astype(jnp.int32).reshape(-1)
    v_flat = A_edge_values.astype(jnp.float32).reshape(-1)

    a = _build_adjacency_call()(r_flat, c_flat, v_flat).reshape(NT, N, N)

    filt = jax.nn.softmax(
        jnp.stack([conv_weights[2], conv_weights[1], conv_weights[0]]), axis=-1)

    xw_aug = _xw_call(X, gcn_weight)
    v = _chain_call(filt, a, xw_aug)
    y = _fin_call(v, gcn_bias.reshape(1, W_OUT), lin_weight,
                  lin_bias.reshape(1, NCLS),
                  target_x.astype(jnp.int32).reshape(NTGT, 1))
    return y


# speedup vs baseline: 3.8969x; 1.3582x over previous
import functools

import jax
import jax.numpy as jnp
from jax import lax
from jax.experimental import pallas as pl
from jax.experimental.pallas import tpu as pltpu
from jax.experimental.pallas import tpu_sc as plsc

N = 2048
NT = 4
E = 32768
QR = 256
NQ = N // QR
NSC = 2
NTILE = 16
EPT = E // NTILE
PANEL = QR * N
SLICE = PANEL // NTILE

W_IN = 256
W_OUT = 128
NCH = 2
NCLS = 4
NTGT = 1024
KB = 256
NKB = N // KB


def _adj_body(r_hbm, c_hbm, v_hbm, o0, o1, o2, o3,
              re, ce, ve, idxb, valb, zbuf, acc):
    outs = (o0, o1, o2, o3)
    cid = lax.axis_index("c")
    sid = lax.axis_index("s")

    def zb(i, carry):
        zbuf[pl.ds(i * 16, 16)] = jnp.zeros((16,), jnp.float32)
        return carry

    lax.fori_loop(0, SLICE // 16, zb, 0)

    rounds_per_type = NQ // NSC
    for rd in range(NT * rounds_per_type):
        t = rd // rounds_per_type
        q = NSC * (rd % rounds_per_type) + cid
        lo = q * QR

        pltpu.sync_copy(zbuf, acc.at[pl.ds(sid * SLICE, SLICE)])

        if rd % rounds_per_type == 0:
            ebase = t * E + sid * EPT
            pltpu.sync_copy(r_hbm.at[pl.ds(ebase, EPT)], re)
            pltpu.sync_copy(c_hbm.at[pl.ds(ebase, EPT)], ce)
            pltpu.sync_copy(v_hbm.at[pl.ds(ebase, EPT)], ve)

        def body(i, carry):
            sl = pl.ds(i * 16, 16)
            r = re[sl]
            c = ce[sl]
            v = ve[sl]
            inq = (r >= lo) & (r < lo + QR)
            idxb[sl] = ((c >> 7) << 15) + ((r & (QR - 1)) << 7) + (c & 127)
            valb[sl] = jnp.where(inq, v, jnp.zeros_like(v))
            return carry

        lax.fori_loop(0, EPT // 16, body, 0)

        plsc.subcore_barrier()
        pltpu.sync_copy(valb, acc.at[idxb], add=True)
        plsc.subcore_barrier()

        base = q * PANEL + sid * SLICE
        pltpu.sync_copy(acc.at[pl.ds(sid * SLICE, SLICE)],
                        outs[t].at[pl.ds(base, SLICE)])


@functools.cache
def _build_adjacency_call():
    mesh = plsc.VectorSubcoreMesh(core_axis_name="c", subcore_axis_name="s",
                                  num_cores=NSC, num_subcores=NTILE)
    return pl.kernel(
        _adj_body,
        out_type=tuple(jax.ShapeDtypeStruct((N * N,), jnp.float32)
                       for _ in range(NT)),
        mesh=mesh,
        scratch_types=[
            pltpu.VMEM((EPT,), jnp.int32),
            pltpu.VMEM((EPT,), jnp.int32),
            pltpu.VMEM((EPT,), jnp.float32),
            pltpu.VMEM((EPT,), jnp.int32),
            pltpu.VMEM((EPT,), jnp.float32),
            pltpu.VMEM((SLICE,), jnp.float32),
            pltpu.VMEM_SHARED((PANEL,), jnp.float32),
        ],
    )


def _xw_body(x_ref, w_ref, out_ref):
    xw = jnp.dot(x_ref[...], w_ref[...], preferred_element_type=jnp.float32)
    out_ref[:, 0:W_OUT] = xw
    i = lax.broadcasted_iota(jnp.int32, (N, 128), 1)
    out_ref[:, W_OUT:W_OUT + 128] = jnp.where(i == 0, 1.0, 0.0)


_xw_call = pl.pallas_call(
    _xw_body,
    out_shape=jax.ShapeDtypeStruct((N, 256), jnp.float32),
)


def _chain_body(filt_ref, a0, a1, a2, a3, xw_ref, out_ref, buf0, buf1, mscr):
    s = pl.program_id(0)
    q = pl.program_id(1)
    arefs = (a0, a1, a2, a3)
    rows = pl.ds(pl.multiple_of(q * QR, QR), QR)

    def stage(si, in_at, store):
        for c in range(NCH):
            m = None
            for e in range(NT):
                term = arefs[e][...] * filt_ref[si, c, e]
                m = term if m is None else m + term
            mscr[...] = m
            acc = None
            for ch in range(N // 128):
                mm = mscr[ch * QR:(ch + 1) * QR, :]
                p = jnp.dot(mm, in_at(c, ch),
                            preferred_element_type=jnp.float32)
                acc = p if acc is None else acc + p
            store(c, acc)

    @pl.when(s == 0)
    def _():
        def store(c, val):
            buf0[c, rows, :] = val
        stage(0, lambda c, ch: xw_ref[ch * 128:(ch + 1) * 128, :], store)

    @pl.when(s == 1)
    def _():
        def store(c, val):
            buf1[c, rows, :] = val
        stage(1, lambda c, ch: buf0[c, ch * 128:(ch + 1) * 128, :], store)

    @pl.when(s == 2)
    def _():
        def store(c, val):
            out_ref[c, rows, :] = val
        stage(2, lambda c, ch: buf1[c, ch * 128:(ch + 1) * 128, :], store)


_a_spec = pl.BlockSpec((N // 128 * QR, 128), lambda s, q: (q, 0))

_chain_call = pl.pallas_call(
    _chain_body,
    grid=(3, NQ),
    in_specs=[
        pl.BlockSpec(memory_space=pltpu.SMEM),
        _a_spec, _a_spec, _a_spec, _a_spec,
        pl.BlockSpec((N, 256), lambda s, q: (0, 0)),
    ],
    out_specs=pl.BlockSpec((NCH, N, 256), lambda s, q: (0, 0, 0)),
    out_shape=jax.ShapeDtypeStruct((NCH, N, 256), jnp.float32),
    scratch_shapes=[
        pltpu.VMEM((NCH, N, 256), jnp.float32),
        pltpu.VMEM((NCH, N, 256), jnp.float32),
        pltpu.VMEM((N // 128 * QR, 128), jnp.float32),
    ],
    compiler_params=pltpu.CompilerParams(
        dimension_semantics=("arbitrary", "arbitrary")),
)


def _fin_body(v_ref, bias_ref, lw_ref, lb_ref, tgt_ref, out_ref):
    logits = None
    for c in range(NCH):
        vc = v_ref[c]
        deg = vc[:, W_OUT:W_OUT + 1]
        winv = jnp.where(deg == 0.0, 0.0, 1.0 / deg)
        xc = jnp.maximum(vc[:, 0:W_OUT] * winv + bias_ref[...], 0.0)
        lc = jnp.dot(xc, lw_ref[c * W_OUT:(c + 1) * W_OUT, :],
                     preferred_element_type=jnp.float32)
        logits = lc if logits is None else logits + lc
    tgt = tgt_ref[...]
    oh = (tgt == lax.broadcasted_iota(jnp.int32, (NTGT, N), 1))
    y = jnp.dot(oh.astype(jnp.float32), logits,
                preferred_element_type=jnp.float32)
    out_ref[...] = y + lb_ref[...]


_fin_call = pl.pallas_call(
    _fin_body,
    out_shape=jax.ShapeDtypeStruct((NTGT, NCLS), jnp.float32),
)


def kernel(A_edge_indices, A_edge_values, X, target_x, target, conv_weights,
           gcn_weight, gcn_bias, lin_weight, lin_bias):
    r_flat = A_edge_indices[:, 0, :].astype(jnp.int32).reshape(-1)
    c_flat = A_edge_indices[:, 1, :].astype(jnp.int32).reshape(-1)
    v_flat = A_edge_values.astype(jnp.float32).reshape(-1)

    a_parts = _build_adjacency_call()(r_flat, c_flat, v_flat)
    a_mats = [p.reshape(NQ * (N // 128) * QR, 128) for p in a_parts]

    filt = jax.nn.softmax(
        jnp.stack([conv_weights[2], conv_weights[1], conv_weights[0]]), axis=-1)

    xw_aug = _xw_call(X, gcn_weight)
    v = _chain_call(filt, *a_mats, xw_aug)
    y = _fin_call(v, gcn_bias.reshape(1, W_OUT), lin_weight,
                  lin_bias.reshape(1, NCLS),
                  target_x.astype(jnp.int32).reshape(NTGT, 1))
    return y


# speedup vs baseline: 4.2157x; 1.0818x over previous
import functools

import jax
import jax.numpy as jnp
from jax import lax
from jax.experimental import pallas as pl
from jax.experimental.pallas import tpu as pltpu
from jax.experimental.pallas import tpu_sc as plsc

N = 2048
NT = 4
E = 32768
QR = 256
NQ = N // QR
NSC = 2
NTILE = 16
EPT = E // NTILE
PANEL = QR * N
SLICE = PANEL // NTILE

W_IN = 256
W_OUT = 128
NCH = 2
NCLS = 4
NTGT = 1024
KB = 256
NKB = N // KB


def _adj_body(idx_hbm, val_hbm, o0, o1, o2, o3,
              re, ce, ve, idxb, valb, zbuf, acc, dsem):
    outs = (o0, o1, o2, o3)
    cid = lax.axis_index("c")
    sid = lax.axis_index("s")

    def zb(i, carry):
        zbuf[pl.ds(i * 16, 16)] = jnp.zeros((16,), jnp.float32)
        return carry

    lax.fori_loop(0, SLICE // 16, zb, 0)

    rounds_per_type = NQ // NSC
    prev = None
    for rd in range(NT * rounds_per_type):
        t = rd // rounds_per_type
        q = NSC * (rd % rounds_per_type) + cid
        lo = q * QR

        if rd % rounds_per_type == 0:
            esl = pl.ds(sid * EPT, EPT)
            pltpu.sync_copy(idx_hbm.at[t, 0, esl], re)
            pltpu.sync_copy(idx_hbm.at[t, 1, esl], ce)
            pltpu.sync_copy(val_hbm.at[t, esl], ve)

        def body(i, carry):
            sl = pl.ds(i * 16, 16)
            r = re[sl]
            c = ce[sl]
            v = ve[sl]
            inq = (r >= lo) & (r < lo + QR)
            idxb[sl] = ((c >> 7) << 15) + ((r & (QR - 1)) << 7) + (c & 127)
            valb[sl] = jnp.where(inq, v, jnp.zeros_like(v))
            return carry

        lax.fori_loop(0, EPT // 16, body, 0)

        if prev is not None:
            prev.wait()
        pltpu.sync_copy(zbuf, acc.at[pl.ds(sid * SLICE, SLICE)])

        plsc.subcore_barrier()
        pltpu.sync_copy(valb, acc.at[idxb], add=True)
        plsc.subcore_barrier()

        base = q * PANEL + sid * SLICE
        prev = pltpu.async_copy(acc.at[pl.ds(sid * SLICE, SLICE)],
                                outs[t].at[pl.ds(base, SLICE)], dsem)
    prev.wait()


@functools.cache
def _build_adjacency_call():
    mesh = plsc.VectorSubcoreMesh(core_axis_name="c", subcore_axis_name="s",
                                  num_cores=NSC, num_subcores=NTILE)
    return pl.kernel(
        _adj_body,
        out_type=tuple(jax.ShapeDtypeStruct((N * N,), jnp.float32)
                       for _ in range(NT)),
        mesh=mesh,
        scratch_types=[
            pltpu.VMEM((EPT,), jnp.int32),
            pltpu.VMEM((EPT,), jnp.int32),
            pltpu.VMEM((EPT,), jnp.float32),
            pltpu.VMEM((EPT,), jnp.int32),
            pltpu.VMEM((EPT,), jnp.float32),
            pltpu.VMEM((SLICE,), jnp.float32),
            pltpu.VMEM_SHARED((PANEL,), jnp.float32),
            pltpu.SemaphoreType.DMA,
        ],
    )


def _xw_body(x_ref, w_ref, out_ref):
    xw = jnp.dot(x_ref[...], w_ref[...], preferred_element_type=jnp.float32)
    out_ref[:, 0:W_OUT] = xw
    i = lax.broadcasted_iota(jnp.int32, (N, 128), 1)
    out_ref[:, W_OUT:W_OUT + 128] = jnp.where(i == 0, 1.0, 0.0)


_xw_call = pl.pallas_call(
    _xw_body,
    out_shape=jax.ShapeDtypeStruct((N, 256), jnp.float32),
)


def _chain_body(filt_ref, a0, a1, a2, a3, xw_ref, out_ref, buf0, buf1, mscr):
    s = pl.program_id(0)
    q = pl.program_id(1)
    arefs = (a0, a1, a2, a3)
    rows = pl.ds(pl.multiple_of(q * QR, QR), QR)

    def stage(si, in_at, store):
        for c in range(NCH):
            m = None
            for e in range(NT):
                term = arefs[e][...] * filt_ref[si, c, e]
                m = term if m is None else m + term
            mscr[...] = m
            acc = None
            for ch in range(N // 128):
                mm = mscr[ch * QR:(ch + 1) * QR, :]
                p = jnp.dot(mm, in_at(c, ch),
                            preferred_element_type=jnp.float32)
                acc = p if acc is None else acc + p
            store(c, acc)

    @pl.when(s == 0)
    def _():
        def store(c, val):
            buf0[c, rows, :] = val
        stage(0, lambda c, ch: xw_ref[ch * 128:(ch + 1) * 128, :], store)

    @pl.when(s == 1)
    def _():
        def store(c, val):
            buf1[c, rows, :] = val
        stage(1, lambda c, ch: buf0[c, ch * 128:(ch + 1) * 128, :], store)

    @pl.when(s == 2)
    def _():
        def store(c, val):
            out_ref[c, rows, :] = val
        stage(2, lambda c, ch: buf1[c, ch * 128:(ch + 1) * 128, :], store)


_a_spec = pl.BlockSpec((N // 128 * QR, 128), lambda s, q: (q, 0))

_chain_call = pl.pallas_call(
    _chain_body,
    grid=(3, NQ),
    in_specs=[
        pl.BlockSpec(memory_space=pltpu.SMEM),
        _a_spec, _a_spec, _a_spec, _a_spec,
        pl.BlockSpec((N, 256), lambda s, q: (0, 0)),
    ],
    out_specs=pl.BlockSpec((NCH, N, 256), lambda s, q: (0, 0, 0)),
    out_shape=jax.ShapeDtypeStruct((NCH, N, 256), jnp.float32),
    scratch_shapes=[
        pltpu.VMEM((NCH, N, 256), jnp.float32),
        pltpu.VMEM((NCH, N, 256), jnp.float32),
        pltpu.VMEM((N // 128 * QR, 128), jnp.float32),
    ],
    compiler_params=pltpu.CompilerParams(
        dimension_semantics=("arbitrary", "arbitrary")),
)


def _fin_body(v_ref, bias_ref, lw_ref, lb_ref, tgt_ref, out_ref):
    logits = None
    for c in range(NCH):
        vc = v_ref[c]
        deg = vc[:, W_OUT:W_OUT + 1]
        winv = jnp.where(deg == 0.0, 0.0, 1.0 / deg)
        xc = jnp.maximum(vc[:, 0:W_OUT] * winv + bias_ref[...], 0.0)
        lc = jnp.dot(xc, lw_ref[c * W_OUT:(c + 1) * W_OUT, :],
                     preferred_element_type=jnp.float32)
        logits = lc if logits is None else logits + lc
    tgt = tgt_ref[...]
    oh = (tgt == lax.broadcasted_iota(jnp.int32, (NTGT, N), 1))
    y = jnp.dot(oh.astype(jnp.float32), logits,
                preferred_element_type=jnp.float32)
    out_ref[...] = y + lb_ref[...]


_fin_call = pl.pallas_call(
    _fin_body,
    out_shape=jax.ShapeDtypeStruct((NTGT, NCLS), jnp.float32),
)


def kernel(A_edge_indices, A_edge_values, X, target_x, target, conv_weights,
           gcn_weight, gcn_bias, lin_weight, lin_bias):
    a_parts = _build_adjacency_call()(A_edge_indices.astype(jnp.int32),
                                      A_edge_values.astype(jnp.float32))
    a_mats = [p.reshape(NQ * (N // 128) * QR, 128) for p in a_parts]

    filt = jax.nn.softmax(
        jnp.stack([conv_weights[2], conv_weights[1], conv_weights[0]]), axis=-1)

    xw_aug = _xw_call(X, gcn_weight)
    v = _chain_call(filt, *a_mats, xw_aug)
    y = _fin_call(v, gcn_bias.reshape(1, W_OUT), lin_weight,
                  lin_bias.reshape(1, NCLS),
                  target_x.astype(jnp.int32).reshape(NTGT, 1))
    return y
